# baseline (speedup 1.0000x reference)
import functools

import jax
import jax.numpy as jnp
from jax import lax
from jax.experimental import pallas as pl
from jax.experimental.pallas import tpu as pltpu
from jax.experimental.pallas import tpu_sc as plsc

B = 16384
D = 8
NC = 2
NS = 16
NW = NC * NS
BPW = B // NW
CHUNK = 128
NCHUNK = BPW // CHUNK


def _sc_gather_body(uidx_hbm, iidx_hbm, t_umlp, t_imlp, t_umf, t_imf,
                    o_umlp, o_imlp, o_umf, o_imf,
                    uidx_v, iidx_v, r_umlp, r_imlp, r_umf, r_imf, sem):
    wid = lax.axis_index("s") * NC + lax.axis_index("c")
    base = wid * BPW
    for j in range(NCHUNK):
        pltpu.sync_copy(uidx_hbm.at[pl.ds(base + j * CHUNK, CHUNK)], uidx_v.at[j])
        pltpu.sync_copy(iidx_hbm.at[pl.ds(base + j * CHUNK, CHUNK)], iidx_v.at[j])
    copies = []
    for j in range(NCHUNK):
        sl = pl.ds(j * CHUNK, CHUNK)
        copies.append(pltpu.async_copy(t_umlp.at[uidx_v.at[j]], r_umlp.at[sl], sem))
        copies.append(pltpu.async_copy(t_imlp.at[iidx_v.at[j]], r_imlp.at[sl], sem))
        copies.append(pltpu.async_copy(t_umf.at[uidx_v.at[j]], r_umf.at[sl], sem))
        copies.append(pltpu.async_copy(t_imf.at[iidx_v.at[j]], r_imf.at[sl], sem))
    for c in copies:
        c.wait()
    pltpu.sync_copy(r_umlp, o_umlp.at[pl.ds(base, BPW)])
    pltpu.sync_copy(r_imlp, o_imlp.at[pl.ds(base, BPW)])
    pltpu.sync_copy(r_umf, o_umf.at[pl.ds(base, BPW)])
    pltpu.sync_copy(r_imf, o_imf.at[pl.ds(base, BPW)])


_sc_gather = functools.partial(
    pl.kernel,
    out_type=[jax.ShapeDtypeStruct((B, D), jnp.float32)] * 4,
    mesh=plsc.VectorSubcoreMesh(core_axis_name="c", subcore_axis_name="s"),
    compiler_params=pltpu.CompilerParams(use_tc_tiling_on_sc=False),
    scratch_types=[
        pltpu.VMEM((NCHUNK, CHUNK), jnp.int32),
        pltpu.VMEM((NCHUNK, CHUNK), jnp.int32),
        pltpu.VMEM((BPW, D), jnp.float32),
        pltpu.VMEM((BPW, D), jnp.float32),
        pltpu.VMEM((BPW, D), jnp.float32),
        pltpu.VMEM((BPW, D), jnp.float32),
        pltpu.SemaphoreType.DMA,
    ],
)(_sc_gather_body)


BLK = 2048


def _tc_mlp_body(u_mlp, i_mlp, u_mf, i_mf,
                 w0u, w0i, b0, w1t, b1, w2t, b2, wa_mlp, wa_mf, ba,
                 out):
    h = u_mlp[...] @ w0u[...] + i_mlp[...] @ w0i[...] + b0[...]
    h = jnp.maximum(h, 0.0)
    h = jnp.maximum(h @ w1t[...] + b1[...], 0.0)
    h = jnp.maximum(h @ w2t[...] + b2[...], 0.0)
    mf = u_mf[...] * i_mf[...]
    out[...] = h @ wa_mlp[...] + mf @ wa_mf[...] + ba[...]


def _full(shape):
    return pl.BlockSpec(shape, lambda i: (0,) * len(shape))


def kernel(user_indices, item_indices, emb_user_mlp, emb_item_mlp,
           emb_user_mf, emb_item_mf, W0, b0, W1, b1, W2, b2, Wa, ba):
    g_umlp, g_imlp, g_umf, g_imf = _sc_gather(
        user_indices, item_indices, emb_user_mlp, emb_item_mlp,
        emb_user_mf, emb_item_mf)

    w0u = W0[:, :D].T
    w0i = W0[:, D:].T
    w1t = W1.T
    w2t = W2.T
    wa_mlp = Wa[:, :8].T
    wa_mf = Wa[:, 8:].T
    b0r = b0.reshape(1, -1)
    b1r = b1.reshape(1, -1)
    b2r = b2.reshape(1, -1)
    bar = ba.reshape(1, -1)

    out = pl.pallas_call(
        _tc_mlp_body,
        grid=(B // BLK,),
        in_specs=[
            pl.BlockSpec((BLK, D), lambda i: (i, 0)),
            pl.BlockSpec((BLK, D), lambda i: (i, 0)),
            pl.BlockSpec((BLK, D), lambda i: (i, 0)),
            pl.BlockSpec((BLK, D), lambda i: (i, 0)),
            _full((D, 32)), _full((D, 32)), _full((1, 32)),
            _full((32, 16)), _full((1, 16)),
            _full((16, 8)), _full((1, 8)),
            _full((8, 1)), _full((8, 1)), _full((1, 1)),
        ],
        out_specs=pl.BlockSpec((BLK, 1), lambda i: (i, 0)),
        out_shape=jax.ShapeDtypeStruct((B, 1), jnp.float32),
    )(g_umlp, g_imlp, g_umf, g_imf,
      w0u, w0i, b0r, w1t, b1r, w2t, b2r, wa_mlp, wa_mf, bar)
    return out
